# Initial kernel scaffold; baseline (speedup 1.0000x reference)
import jax
import jax.numpy as jnp
from jax.experimental import pallas as pl


def kernel(x, edge_index, edge_weight, params):
    raise NotImplementedError("write your pallas kernel here")


# speedup vs baseline: 5.9627x; 5.9627x over previous
import functools

import jax
import jax.numpy as jnp
from jax import lax
from jax.experimental import pallas as pl
from jax.experimental.pallas import tpu as pltpu
from jax.experimental.pallas import tpu_sc as plsc

N = 10000
NP = 10240
E = 320000
NC, NS, L = 2, 16, 16
NW = NC * NS
K = 128
EPW = 10112
E_PAD = NW * EPW
CHUNKS = EPW // K

F32 = jnp.float32


def _make_agg(w):
  rows_per_sub = NP // NS
  zcopies = rows_per_sub // K
  mesh = plsc.VectorSubcoreMesh(core_axis_name="c", subcore_axis_name="s")

  @functools.partial(
      pl.kernel,
      out_type=jax.ShapeDtypeStruct((NC, NP, w), F32),
      mesh=mesh,
      compiler_params=pltpu.CompilerParams(use_tc_tiling_on_sc=False,
                                           needs_layout_passes=False),
      scratch_types=[
          pltpu.VMEM((K,), jnp.int32),
          pltpu.VMEM((K,), jnp.int32),
          pltpu.VMEM((K,), F32),
          pltpu.VMEM((N,), F32),
          pltpu.VMEM((K, w), F32),
          pltpu.VMEM_SHARED((NP, w), F32),
      ],
  )
  def agg(g_hbm, src_hbm, dst_hbm, ew_hbm, dinv_hbm, out_hbm,
          src_v, dst_v, ew_v, dinv_v, rows_v, acc_sh):
    c = lax.axis_index("c")
    s = lax.axis_index("s")
    wid = c * NS + s

    pltpu.sync_copy(dinv_hbm, dinv_v)

    def zrow(i, _):
      for j in range(w // L):
        rows_v[i, pl.ds(j * L, L)] = jnp.zeros((L,), F32)
      return 0
    lax.fori_loop(0, K, zrow, 0)
    for i in range(zcopies):
      pltpu.sync_copy(rows_v, acc_sh.at[pl.ds(s * rows_per_sub + i * K, K)])
    plsc.subcore_barrier()

    def chunk(gi, _):
      base = wid * EPW + gi * K
      pltpu.sync_copy(src_hbm.at[pl.ds(base, K)], src_v)
      pltpu.sync_copy(dst_hbm.at[pl.ds(base, K)], dst_v)
      pltpu.sync_copy(ew_hbm.at[pl.ds(base, K)], ew_v)
      pltpu.sync_copy(g_hbm.at[src_v], rows_v)
      def egroup(g16, _):
        sl16 = pl.ds(g16 * L, L)
        s16 = src_v[sl16]
        d16 = dst_v[sl16]
        w16 = ew_v[sl16]
        nv = plsc.load_gather(dinv_v, [s16]) * w16
        nv = nv * plsc.load_gather(dinv_v, [d16])
        for t in range(L):
          nt = nv[t]
          e = g16 * L + t
          for j in range(w // L):
            sl = pl.ds(j * L, L)
            rows_v[e, sl] = rows_v[e, sl] * nt
        return 0
      lax.fori_loop(0, K // L, egroup, 0)
      pltpu.sync_copy(rows_v, acc_sh.at[dst_v], add=True)
      return 0
    lax.fori_loop(0, CHUNKS, chunk, 0)

    plsc.subcore_barrier()
    for i in range(zcopies):
      sl = pl.ds(s * rows_per_sub + i * K, K)
      pltpu.sync_copy(acc_sh.at[sl], out_hbm.at[c, sl])

  return agg


def _bn(h, g, b):
  mu = jnp.mean(h, axis=0, keepdims=True)
  var = jnp.var(h, axis=0, keepdims=True)
  return g * (h - mu) / jnp.sqrt(var + 1e-5) + b


def _tc(fn, out_shape, *args):
  return pl.pallas_call(fn, out_shape=out_shape)(*args)


DIMS = [(128, 128), (128, 128), (128, 64), (64, 32), (32, 64), (64, 128),
        (128, 64), (64, 32)]
AGG_W = [do for _, do in DIMS]


def kernel(x, edge_index, edge_weight, params):
  p = params
  src = edge_index[0]
  dst = edge_index[1]
  pad = E_PAD - E
  srcp = jnp.pad(src, (0, pad))
  dstp = jnp.pad(dst, (0, pad))
  ewp = jnp.pad(edge_weight, (0, pad))

  aggs = {w: _make_agg(w) for w in sorted(set(AGG_W) | {16})}

  ones16 = jnp.ones((N, 16), F32)
  onesN = jnp.ones((N,), F32)
  s0 = aggs[16](ones16, srcp, dstp, ewp, onesN)

  def tc0(x_ref, w1_ref, g_ref, be_ref, s0_ref, dinv_ref, dinv2_ref, a1_ref):
    deg = (s0_ref[0, :N, 0:1] + s0_ref[1, :N, 0:1]) + 1.0
    dinv = lax.rsqrt(deg)
    dinv_ref[...] = dinv
    dinv2_ref[...] = dinv * dinv
    h = _bn(x_ref[...], g_ref[...], be_ref[...])
    a1_ref[...] = jnp.dot(h, w1_ref[...], preferred_element_type=F32)

  dinv, dinv2, a = _tc(
      tc0,
      (jax.ShapeDtypeStruct((N, 1), F32),
       jax.ShapeDtypeStruct((N, 1), F32),
       jax.ShapeDtypeStruct((N, AGG_W[0]), F32)),
      x, p["W1"], p["g1"], p["be1"], s0)

  dinv_flat = dinv.reshape((N,))

  for i in range(1, 9):
    wi = AGG_W[i - 1]
    s_i = aggs[wi](a, srcp, dstp, ewp, dinv_flat)

    if i < 8:
      wn = AGG_W[i]

      def tci(s_ref, a_ref, dinv2_ref, b_ref, g_ref, be_ref,
              wn_ref, an_ref):
        conv = (s_ref[0, :N, :] + s_ref[1, :N, :]) + a_ref[...] * dinv2_ref[...]
        r = conv + b_ref[...]
        h = jax.nn.relu(_bn(r, g_ref[...], be_ref[...]))
        an_ref[...] = jnp.dot(h, wn_ref[...], preferred_element_type=F32)

      a = _tc(
          tci, jax.ShapeDtypeStruct((N, wn), F32),
          s_i, a, dinv2, p["b%d" % i],
          p["g%d" % (i + 1)], p["be%d" % (i + 1)], p["W%d" % (i + 1)])
    else:
      def tc8(s_ref, a_ref, dinv2_ref, b_ref, g9_ref, be9_ref,
              lw1_ref, lb1_ref, g10_ref, be10_ref, lw2_ref, lb2_ref,
              out_ref):
        conv = (s_ref[0, :N, :] + s_ref[1, :N, :]) + a_ref[...] * dinv2_ref[...]
        h = _bn(conv + b_ref[...], g9_ref[...], be9_ref[...])
        t1 = jnp.dot(jax.nn.relu(h), lw1_ref[...],
                     preferred_element_type=F32) + lb1_ref[...]
        t1 = _bn(t1, g10_ref[...], be10_ref[...])
        out_ref[...] = jnp.dot(jax.nn.relu(t1), lw2_ref[...],
                               preferred_element_type=F32) + lb2_ref[...]

      out = _tc(
          tc8, jax.ShapeDtypeStruct((N, 40), F32),
          s_i, a, dinv2, p["b8"], p["g9"], p["be9"],
          p["lw1"], p["lb1"], p["g10"], p["be10"], p["lw2"], p["lb2"])

  return out
